# baseline (speedup 1.0000x reference)
import functools

import jax
import jax.numpy as jnp
from jax import lax
from jax.experimental import pallas as pl
from jax.experimental.pallas import tpu as pltpu
from jax.experimental.pallas import tpu_sc as plsc

N = 100000
NP = 100096
E = 1600000
DH = 16
NC = 2
NS = 16
B = 80
G = 5
GB = G * B
EPT = E // NS
NCH = EPT // B
NG = NCH // G
NQ = (NG - 2) // 4
RPT = NP // NS
WB = 368
NWB = RPT // WB
NROW = E // B

_mesh = plsc.VectorSubcoreMesh(
    core_axis_name="c", subcore_axis_name="s", num_cores=NC, num_subcores=NS
)


@functools.partial(
    pl.kernel,
    out_type=jax.ShapeDtypeStruct((NC, NP, DH), jnp.float32),
    mesh=_mesh,
    scratch_types=[
        pltpu.VMEM((4 * G, B), jnp.int32),
        pltpu.VMEM((4 * G, B), jnp.int32),
        pltpu.VMEM((GB, DH), jnp.float32),
        pltpu.VMEM((GB, DH), jnp.float32),
        pltpu.VMEM((GB, DH), jnp.float32),
        pltpu.VMEM((GB, DH), jnp.float32),
        pltpu.VMEM_SHARED((NP, DH), jnp.float32),
        pltpu.SemaphoreType.DMA,
        pltpu.SemaphoreType.DMA,
        pltpu.SemaphoreType.DMA,
        pltpu.SemaphoreType.DMA,
    ],
    compiler_params=pltpu.CompilerParams(use_tc_tiling_on_sc=False),
)
def _vector_graph_sc(xh, kidx, out, ibig, jbig, bi0, bi1, bj0, bj1,
                     acc, semi, semg, sems0, sems1):
    ii = kidx.at[0]
    jj = kidx.at[1]
    c = lax.axis_index("c")
    s = lax.axis_index("s")
    bis = (bi0, bi1)
    bjs = (bj0, bj1)
    semss = (sems0, sems1)

    zero_v = jnp.zeros((DH,), jnp.float32)

    def zfill(k, carry):
        r0 = k * 8
        for u in range(8):
            bi0[r0 + u, :] = zero_v
        return carry

    lax.fori_loop(0, WB // 8, zfill, 0)

    def zero_body(k, carry):
        row0 = s * RPT + k * WB
        pltpu.sync_copy(bi0.at[pl.ds(0, WB)], acc.at[pl.ds(row0, WB)])
        return carry

    lax.fori_loop(0, NWB, zero_body, 0)
    plsc.subcore_barrier()

    def idx_row0(g):
        return jnp.minimum(s * NCH + g * G, NROW - G)

    def fire_idx_load(g, ph):
        r0 = idx_row0(g)
        pltpu.async_copy(ii.at[pl.ds(r0, G)], ibig.at[pl.ds(ph * G, G)], semi)
        pltpu.async_copy(jj.at[pl.ds(r0, G)], jbig.at[pl.ds(ph * G, G)], semi)

    def wait_idx_load():
        pltpu.make_async_copy(
            ii.at[pl.ds(0, G)], ibig.at[pl.ds(0, G)], semi).wait()
        pltpu.make_async_copy(
            jj.at[pl.ds(0, G)], jbig.at[pl.ds(0, G)], semi).wait()

    def drain_scatters(p):
        pltpu.make_async_copy(
            xh.at[c].at[pl.ds(0, GB)], bis[p], semss[p]).wait()
        pltpu.make_async_copy(
            xh.at[c].at[pl.ds(0, GB)], bjs[p], semss[p]).wait()

    def do_group(g, p, ph, drain):
        bi = bis[p]
        bj = bjs[p]
        sems = semss[p]
        if drain:
            drain_scatters(p)
        wait_idx_load()
        gds = []
        for t in range(G):
            r = ph * G + t
            gds.append(pltpu.async_copy(
                xh.at[c].at[jbig.at[r]], bi.at[pl.ds(t * B, B)], semg))
            gds.append(pltpu.async_copy(
                xh.at[c].at[ibig.at[r]], bj.at[pl.ds(t * B, B)], semg))
        fire_idx_load(g + 2, (ph + 2) % 4)
        for t in range(G):
            r = ph * G + t
            gds[2 * t].wait()
            gds[2 * t + 1].wait()
            pltpu.async_copy(
                bi.at[pl.ds(t * B, B)], acc.at[ibig.at[r]], sems, add=True)
            pltpu.async_copy(
                bj.at[pl.ds(t * B, B)], acc.at[jbig.at[r]], sems, add=True)

    fire_idx_load(0, 0)
    fire_idx_load(1, 1)
    do_group(0, 0, 0, drain=False)
    do_group(1, 1, 1, drain=False)

    def body(m, carry):
        g = 2 + m * 4
        for b in range(4):
            do_group(g + b, b % 2, (2 + b) % 4, drain=True)
        return carry

    lax.fori_loop(0, NQ, body, 0)

    drain_scatters(0)
    drain_scatters(1)
    wait_idx_load()
    wait_idx_load()
    plsc.subcore_barrier()

    def wb_body(k, carry):
        row0 = s * RPT + k * WB
        ca = pltpu.async_copy(acc.at[pl.ds(row0, WB)], bi0.at[pl.ds(0, WB)],
                              semi)
        cx = pltpu.async_copy(xh.at[c].at[pl.ds(row0, WB)],
                              bj0.at[pl.ds(0, WB)], semg)
        ca.wait()
        cx.wait()

        def comp(q, carry2):
            r0 = q * 8
            for u in range(8):
                r = r0 + u
                a = bi0[r, :]
                bi0[r, :] = bj0[r, :] * a[12] - a
            return carry2

        lax.fori_loop(0, WB // 8, comp, 0)
        co = pltpu.async_copy(bi0.at[pl.ds(0, WB)],
                              out.at[c, pl.ds(row0, WB)], sems0)
        co.wait()
        return carry

    lax.fori_loop(0, NWB, wb_body, 0)


def kernel(x, iInd, jInd):
    nb, f1, f2, n = x.shape
    feats = f1 * f2
    half = feats // 2
    xT = x.reshape(NC, half, n).transpose(0, 2, 1)
    xh = jnp.pad(xT, ((0, 0), (0, NP - n), (0, DH - half)))
    xh = xh.at[:, :n, half].set(1.0)
    kidx = jnp.stack([iInd.astype(jnp.int32).reshape(E // B, B),
                      jInd.astype(jnp.int32).reshape(E // B, B)])
    out2 = _vector_graph_sc(xh, kidx)
    return out2[:, :n, :half].transpose(0, 2, 1).reshape(nb, f1, f2, n)


# speedup vs baseline: 1.4724x; 1.4724x over previous
import functools

import jax
import jax.numpy as jnp
from jax import lax
from jax.experimental import pallas as pl
from jax.experimental.pallas import tpu as pltpu
from jax.experimental.pallas import tpu_sc as plsc

N = 100000
NP = 100096
E = 1600000
DH = 16
NC = 2
NS = 16
B = 80
G = 5
GB = G * B
EPT = E // NS
NCH = EPT // B
NG = NCH // G
NQ = (NG - 2) // 4
RPT = NP // NS
WB = 368
NWB = RPT // WB
NROW = E // B

_mesh = plsc.VectorSubcoreMesh(
    core_axis_name="c", subcore_axis_name="s", num_cores=NC, num_subcores=NS
)


@functools.partial(
    pl.kernel,
    out_type=jax.ShapeDtypeStruct((NC, NP, DH), jnp.float32),
    mesh=_mesh,
    scratch_types=[
        pltpu.VMEM((4 * G, B), jnp.int32),
        pltpu.VMEM((4 * G, B), jnp.int32),
        pltpu.VMEM((GB, DH), jnp.float32),
        pltpu.VMEM((GB, DH), jnp.float32),
        pltpu.VMEM((GB, DH), jnp.float32),
        pltpu.VMEM((GB, DH), jnp.float32),
        pltpu.VMEM_SHARED((NP, DH), jnp.float32),
        pltpu.SemaphoreType.DMA,
        pltpu.SemaphoreType.DMA,
        pltpu.SemaphoreType.DMA,
        pltpu.SemaphoreType.DMA,
    ],
    compiler_params=pltpu.CompilerParams(use_tc_tiling_on_sc=False),
)
def _vector_graph_sc(xh, ii, jj, out, ibig, jbig, bi0, bi1, bj0, bj1,
                     acc, semi, semg, sems0, sems1):
    c = lax.axis_index("c")
    s = lax.axis_index("s")
    bis = (bi0, bi1)
    bjs = (bj0, bj1)
    semss = (sems0, sems1)

    zero_v = jnp.zeros((DH,), jnp.float32)

    def zfill(k, carry):
        r0 = k * 8
        for u in range(8):
            bi0[r0 + u, :] = zero_v
        return carry

    lax.fori_loop(0, WB // 8, zfill, 0)

    def zero_body(k, carry):
        row0 = s * RPT + k * WB
        pltpu.sync_copy(bi0.at[pl.ds(0, WB)], acc.at[pl.ds(row0, WB)])
        return carry

    lax.fori_loop(0, NWB, zero_body, 0)
    plsc.subcore_barrier()

    def idx_row0(g):
        return jnp.minimum(s * NCH + g * G, NROW - G)

    def fire_idx_load(g, ph):
        r0 = idx_row0(g)
        pltpu.async_copy(ii.at[pl.ds(r0, G)], ibig.at[pl.ds(ph * G, G)], semi)
        pltpu.async_copy(jj.at[pl.ds(r0, G)], jbig.at[pl.ds(ph * G, G)], semi)

    def wait_idx_load():
        pltpu.make_async_copy(
            ii.at[pl.ds(0, G)], ibig.at[pl.ds(0, G)], semi).wait()
        pltpu.make_async_copy(
            jj.at[pl.ds(0, G)], jbig.at[pl.ds(0, G)], semi).wait()

    def drain_scatters(p):
        pltpu.make_async_copy(
            xh.at[c].at[pl.ds(0, GB)], bis[p], semss[p]).wait()
        pltpu.make_async_copy(
            xh.at[c].at[pl.ds(0, GB)], bjs[p], semss[p]).wait()

    def do_group(g, p, ph, drain):
        bi = bis[p]
        bj = bjs[p]
        sems = semss[p]
        if drain:
            drain_scatters(p)
        wait_idx_load()
        gds = []
        for t in range(G):
            r = ph * G + t
            gds.append(pltpu.async_copy(
                xh.at[c].at[jbig.at[r]], bi.at[pl.ds(t * B, B)], semg))
            gds.append(pltpu.async_copy(
                xh.at[c].at[ibig.at[r]], bj.at[pl.ds(t * B, B)], semg))
        fire_idx_load(g + 2, (ph + 2) % 4)
        for t in range(G):
            r = ph * G + t
            gds[2 * t].wait()
            gds[2 * t + 1].wait()
            pltpu.async_copy(
                bi.at[pl.ds(t * B, B)], acc.at[ibig.at[r]], sems, add=True)
            pltpu.async_copy(
                bj.at[pl.ds(t * B, B)], acc.at[jbig.at[r]], sems, add=True)

    fire_idx_load(0, 0)
    fire_idx_load(1, 1)
    do_group(0, 0, 0, drain=False)
    do_group(1, 1, 1, drain=False)

    def body(m, carry):
        g = 2 + m * 4
        for b in range(4):
            do_group(g + b, b % 2, (2 + b) % 4, drain=True)
        return carry

    lax.fori_loop(0, NQ, body, 0)

    drain_scatters(0)
    drain_scatters(1)
    wait_idx_load()
    wait_idx_load()
    plsc.subcore_barrier()

    def wb_body(k, carry):
        row0 = s * RPT + k * WB
        ca = pltpu.async_copy(acc.at[pl.ds(row0, WB)], bi0.at[pl.ds(0, WB)],
                              semi)
        cx = pltpu.async_copy(xh.at[c].at[pl.ds(row0, WB)],
                              bj0.at[pl.ds(0, WB)], semg)
        ca.wait()
        cx.wait()

        def comp(q, carry2):
            r0 = q * 8
            for u in range(8):
                r = r0 + u
                a = bi0[r, :]
                bi0[r, :] = bj0[r, :] * a[12] - a
            return carry2

        lax.fori_loop(0, WB // 8, comp, 0)
        co = pltpu.async_copy(bi0.at[pl.ds(0, WB)],
                              out.at[c, pl.ds(row0, WB)], sems0)
        co.wait()
        return carry

    lax.fori_loop(0, NWB, wb_body, 0)


def kernel(x, iInd, jInd):
    nb, f1, f2, n = x.shape
    feats = f1 * f2
    half = feats // 2
    xT = x.reshape(feats, n).T
    ones = jnp.ones((n, 1), jnp.float32)
    zpad = jnp.zeros((n, DH - half - 1), jnp.float32)
    xa = jnp.concatenate([xT[:, :half], ones, zpad], axis=1)
    xb = jnp.concatenate([xT[:, half:], ones, zpad], axis=1)
    xh = jnp.pad(jnp.stack([xa, xb]), ((0, 0), (0, NP - n), (0, 0)))
    ii2 = iInd.astype(jnp.int32).reshape(E // B, B)
    jj2 = jInd.astype(jnp.int32).reshape(E // B, B)
    out2 = _vector_graph_sc(xh, ii2, jj2)
    o = jnp.concatenate([out2[0, :n, :half], out2[1, :n, :half]], axis=1)
    return o.T.reshape(nb, f1, f2, n)
